# baseline (speedup 1.0000x reference)
import functools

import jax
import jax.numpy as jnp
from jax import lax
from jax.experimental import pallas as pl
from jax.experimental.pallas import tpu as pltpu
from jax.experimental.pallas import tpu_sc as plsc

BATCH = 16384
EMBED = 32
NROWS = 1000001
NUM_CORES = 2
NUM_SUBCORES = 16
NUM_WORKERS = NUM_CORES * NUM_SUBCORES
BPW = BATCH // NUM_WORKERS
GCHUNK = 128
NCHUNK = BPW // GCHUNK

_mesh = plsc.VectorSubcoreMesh(core_axis_name="c", subcore_axis_name="s")


def _detile_body(ut_hbm, it_hbm, *rest):
    outs, sem = rest[:-1], rest[-1]
    window = []
    for d in range(EMBED):
        window.append(pltpu.async_copy(ut_hbm.at[d], outs[d], sem))
        window.append(pltpu.async_copy(it_hbm.at[d], outs[EMBED + d], sem))
        while len(window) > 14:
            window.pop(0).wait()
    for c in window:
        c.wait()


_detile = pl.pallas_call(
    _detile_body,
    grid=(),
    in_specs=[pl.BlockSpec(memory_space=pltpu.MemorySpace.HBM)] * 2,
    out_specs=[pl.BlockSpec(memory_space=pltpu.MemorySpace.HBM)] * (2 * EMBED),
    out_shape=[jax.ShapeDtypeStruct((NROWS,), jnp.float32)] * (2 * EMBED),
    scratch_shapes=[pltpu.SemaphoreType.DMA],
)


def _proj_body(g_ref, a_ref, gw_ref, gb_ref, aw_ref, ab_ref, pg_ref, pa_ref):
    pg_ref[...] = jnp.dot(gw_ref[...], g_ref[...],
                          preferred_element_type=jnp.float32) + gb_ref[...]
    pa_ref[...] = jnp.dot(aw_ref[...], a_ref[...],
                          preferred_element_type=jnp.float32) + ab_ref[...]


_proj = pl.pallas_call(
    _proj_body,
    out_shape=[
        jax.ShapeDtypeStruct((EMBED, BATCH), jnp.float32),
        jax.ShapeDtypeStruct((EMBED, BATCH), jnp.float32),
    ],
)


@functools.partial(
    pl.kernel,
    out_type=jax.ShapeDtypeStruct((BATCH,), jnp.float32),
    mesh=_mesh,
    scratch_types=[
        pltpu.VMEM((BPW,), jnp.int32),
        pltpu.VMEM((BPW,), jnp.int32),
        pltpu.VMEM((EMBED, BPW), jnp.float32),
        pltpu.VMEM((EMBED, BPW), jnp.float32),
        pltpu.VMEM((EMBED, BPW), jnp.float32),
        pltpu.VMEM((EMBED, BPW), jnp.float32),
        pltpu.VMEM((BPW,), jnp.float32),
        pltpu.SemaphoreType.DMA,
        pltpu.SemaphoreType.DMA,
    ],
    compiler_params=pltpu.CompilerParams(use_tc_tiling_on_sc=False),
)
def _sc_gather_combine(uids_hbm, iids_hbm, *rest):
    tabs = rest[:2 * EMBED]
    pgT_hbm, paT_hbm, out_hbm = rest[2 * EMBED:2 * EMBED + 3]
    (uidx, iidx, vtu, vti, vpa, vpg, vout, gsem, psem) = rest[2 * EMBED + 3:]
    wid = lax.axis_index("s") * NUM_CORES + lax.axis_index("c")
    base = wid * BPW
    cpa = pltpu.async_copy(paT_hbm.at[:, pl.ds(base, BPW)], vpa, psem)
    cpg = pltpu.async_copy(pgT_hbm.at[:, pl.ds(base, BPW)], vpg, psem)
    pltpu.sync_copy(uids_hbm.at[pl.ds(base, BPW)], uidx)
    pltpu.sync_copy(iids_hbm.at[pl.ds(base, BPW)], iidx)
    window = []
    for d in range(EMBED):
        batch = []
        for k in range(NCHUNK):
            sl = pl.ds(k * GCHUNK, GCHUNK)
            batch.append(
                pltpu.async_copy(tabs[d].at[uidx.at[sl]], vtu.at[d, sl],
                                 gsem))
            batch.append(
                pltpu.async_copy(tabs[EMBED + d].at[iidx.at[sl]],
                                 vti.at[d, sl], gsem))
        window.append(batch)
        if len(window) > 2:
            for c in window.pop(0):
                c.wait()
    for batch in window:
        for c in batch:
            c.wait()
    cpa.wait()
    cpg.wait()

    @pl.loop(0, BPW // 16)
    def _grp(g):
        sl = pl.ds(g * 16, 16)
        acc = jnp.zeros((16,), jnp.float32)
        for d in range(EMBED):
            acc += (vtu[d, sl] + vpa[d, sl]) * (vti[d, sl] + vpg[d, sl])
        vout[sl] = 4.0 / (1.0 + jnp.exp(-acc)) + 1.0

    pltpu.sync_copy(vout, out_hbm.at[pl.ds(base, BPW)])


def kernel(user_ids, item_ids, genres_features, age_features, user_table,
           item_table, genres_W, genres_b, age_W, age_b, age_weight,
           genre_weight):
    tabs = _detile(user_table.T, item_table.T)
    gw = (genre_weight[0] * genres_W).T
    gb = (genre_weight[0] * genres_b)[:, None]
    aw = (age_weight[0] * age_W).T
    ab = (age_weight[0] * age_b)[:, None]
    pgT, paT = _proj(genres_features.T, age_features.T, gw, gb, aw, ab)
    return _sc_gather_combine(user_ids, item_ids, *tabs, pgT, paT)


# speedup vs baseline: 6.7186x; 6.7186x over previous
import functools

import jax
import jax.numpy as jnp
from jax import lax
from jax.experimental import pallas as pl
from jax.experimental.pallas import tpu as pltpu
from jax.experimental.pallas import tpu_sc as plsc

BATCH = 16384
EMBED = 32
NROWS = 1000001
TPAN = 8192
NPAN = (NROWS + TPAN - 1) // TPAN
NROWS_PAD = NPAN * TPAN
NUM_CORES = 2
NUM_SUBCORES = 16
NUM_WORKERS = NUM_CORES * NUM_SUBCORES
BPW = BATCH // NUM_WORKERS
GCHUNK = 128
NCHUNK = BPW // GCHUNK

_mesh = plsc.VectorSubcoreMesh(core_axis_name="c", subcore_axis_name="s")


def _transpose_body(u_ref, i_ref, uo_ref, io_ref):
    eye = (lax.broadcasted_iota(jnp.int32, (EMBED, EMBED), 0) ==
           lax.broadcasted_iota(jnp.int32, (EMBED, EMBED), 1)
           ).astype(jnp.float32)
    dn = (((0,), (0,)), ((), ()))
    uo_ref[...] = lax.dot_general(u_ref[...], eye, dn,
                                  preferred_element_type=jnp.float32)
    io_ref[...] = lax.dot_general(i_ref[...], eye, dn,
                                  preferred_element_type=jnp.float32)


_transpose_tables = pl.pallas_call(
    _transpose_body,
    grid=(NPAN,),
    in_specs=[
        pl.BlockSpec((EMBED, TPAN), lambda i: (0, i)),
        pl.BlockSpec((EMBED, TPAN), lambda i: (0, i)),
    ],
    out_specs=[
        pl.BlockSpec((TPAN, EMBED), lambda i: (i, 0)),
        pl.BlockSpec((TPAN, EMBED), lambda i: (i, 0)),
    ],
    out_shape=[
        jax.ShapeDtypeStruct((NROWS_PAD, EMBED), jnp.float32),
        jax.ShapeDtypeStruct((NROWS_PAD, EMBED), jnp.float32),
    ],
)


def _proj_body(g_ref, a_ref, gw_ref, gb_ref, aw_ref, ab_ref, pg_ref, pa_ref):
    dn = (((0,), (0,)), ((), ()))
    pg_ref[...] = lax.dot_general(g_ref[...], gw_ref[...], dn,
                                  preferred_element_type=jnp.float32
                                  ) + gb_ref[...]
    pa_ref[...] = lax.dot_general(a_ref[...], aw_ref[...], dn,
                                  preferred_element_type=jnp.float32
                                  ) + ab_ref[...]


_proj = pl.pallas_call(
    _proj_body,
    out_shape=[
        jax.ShapeDtypeStruct((BATCH, EMBED), jnp.float32),
        jax.ShapeDtypeStruct((BATCH, EMBED), jnp.float32),
    ],
)


@functools.partial(
    pl.kernel,
    out_type=jax.ShapeDtypeStruct((BATCH,), jnp.float32),
    mesh=_mesh,
    scratch_types=[
        pltpu.VMEM((BPW,), jnp.int32),
        pltpu.VMEM((BPW,), jnp.int32),
        pltpu.VMEM((BPW, EMBED), jnp.float32),
        pltpu.VMEM((BPW, EMBED), jnp.float32),
        pltpu.VMEM((BPW, EMBED), jnp.float32),
        pltpu.VMEM((BPW, EMBED), jnp.float32),
        pltpu.VMEM((BPW,), jnp.float32),
        pltpu.SemaphoreType.DMA,
        pltpu.SemaphoreType.DMA,
    ],
    compiler_params=pltpu.CompilerParams(use_tc_tiling_on_sc=False,
                                         needs_layout_passes=False),
)
def _sc_gather_combine(uids_hbm, iids_hbm, utab_hbm, itab_hbm, pg_hbm, pa_hbm,
                       out_hbm, uidx, iidx, vtu, vti, vpg, vpa, vout, gsem,
                       psem):
    wid = lax.axis_index("s") * NUM_CORES + lax.axis_index("c")
    base = wid * BPW
    cpa = pltpu.async_copy(pa_hbm.at[pl.ds(base, BPW)], vpa, psem)
    cpg = pltpu.async_copy(pg_hbm.at[pl.ds(base, BPW)], vpg, psem)
    pltpu.sync_copy(uids_hbm.at[pl.ds(base, BPW)], uidx)
    pltpu.sync_copy(iids_hbm.at[pl.ds(base, BPW)], iidx)
    copies = []
    for k in range(NCHUNK):
        sl = pl.ds(k * GCHUNK, GCHUNK)
        copies.append(
            pltpu.async_copy(utab_hbm.at[uidx.at[sl]], vtu.at[sl], gsem))
        copies.append(
            pltpu.async_copy(itab_hbm.at[iidx.at[sl]], vti.at[sl], gsem))
    for c in copies:
        c.wait()
    cpa.wait()
    cpg.wait()

    @pl.loop(0, BPW // 16)
    def _grp(g):
        row = g * 16 + lax.iota(jnp.int32, 16)
        acc = jnp.zeros((16,), jnp.float32)
        for d in range(EMBED):
            col = jnp.full((16,), d, jnp.int32)
            uv = plsc.load_gather(vtu, [row, col])
            iv = plsc.load_gather(vti, [row, col])
            av = plsc.load_gather(vpa, [row, col])
            gv = plsc.load_gather(vpg, [row, col])
            acc += (uv + av) * (iv + gv)
        vout[pl.ds(g * 16, 16)] = 4.0 / (1.0 + jnp.exp(-acc)) + 1.0

    pltpu.sync_copy(vout, out_hbm.at[pl.ds(base, BPW)])


def kernel(user_ids, item_ids, genres_features, age_features, user_table,
           item_table, genres_W, genres_b, age_W, age_b, age_weight,
           genre_weight):
    utab, itab = _transpose_tables(user_table.T, item_table.T)
    gw = genre_weight[0] * genres_W
    gb = (genre_weight[0] * genres_b)[None, :]
    aw = age_weight[0] * age_W
    ab = (age_weight[0] * age_b)[None, :]
    pg, pa = _proj(genres_features.T, age_features.T, gw, gb, aw, ab)
    return _sc_gather_combine(user_ids, item_ids, utab, itab, pg, pa)


# speedup vs baseline: 6.7466x; 1.0042x over previous
import functools

import jax
import jax.numpy as jnp
from jax import lax
from jax.experimental import pallas as pl
from jax.experimental.pallas import tpu as pltpu
from jax.experimental.pallas import tpu_sc as plsc

BATCH = 16384
EMBED = 32
NROWS = 1000001
TPAN = 8192
NPAN = (NROWS + TPAN - 1) // TPAN
NROWS_PAD = NPAN * TPAN
NUM_CORES = 2
NUM_SUBCORES = 16
NUM_WORKERS = NUM_CORES * NUM_SUBCORES
BPW = BATCH // NUM_WORKERS
GCHUNK = 128
NCHUNK = BPW // GCHUNK

_mesh = plsc.VectorSubcoreMesh(core_axis_name="c", subcore_axis_name="s")


def _transpose_body(u_ref, i_ref, uo_ref, io_ref):
    eye = (lax.broadcasted_iota(jnp.int32, (EMBED, EMBED), 0) ==
           lax.broadcasted_iota(jnp.int32, (EMBED, EMBED), 1)
           ).astype(jnp.bfloat16)
    dn = (((0,), (0,)), ((), ()))
    uo_ref[...] = lax.dot_general(u_ref[...].astype(jnp.bfloat16), eye, dn,
                                  preferred_element_type=jnp.float32)
    io_ref[...] = lax.dot_general(i_ref[...].astype(jnp.bfloat16), eye, dn,
                                  preferred_element_type=jnp.float32)


_transpose_tables = pl.pallas_call(
    _transpose_body,
    grid=(NPAN,),
    compiler_params=pltpu.CompilerParams(fuse_transposed_lhs_in_matmul=True),
    in_specs=[
        pl.BlockSpec((EMBED, TPAN), lambda i: (0, i)),
        pl.BlockSpec((EMBED, TPAN), lambda i: (0, i)),
    ],
    out_specs=[
        pl.BlockSpec((TPAN, EMBED), lambda i: (i, 0)),
        pl.BlockSpec((TPAN, EMBED), lambda i: (i, 0)),
    ],
    out_shape=[
        jax.ShapeDtypeStruct((NROWS_PAD, EMBED), jnp.float32),
        jax.ShapeDtypeStruct((NROWS_PAD, EMBED), jnp.float32),
    ],
)


def _proj_body(g_ref, a_ref, gw_ref, gb_ref, aw_ref, ab_ref, pg_ref, pa_ref):
    dn = (((0,), (0,)), ((), ()))
    pg_ref[...] = lax.dot_general(g_ref[...], gw_ref[...], dn,
                                  preferred_element_type=jnp.float32
                                  ) + gb_ref[...]
    pa_ref[...] = lax.dot_general(a_ref[...], aw_ref[...], dn,
                                  preferred_element_type=jnp.float32
                                  ) + ab_ref[...]


_proj = pl.pallas_call(
    _proj_body,
    out_shape=[
        jax.ShapeDtypeStruct((BATCH, EMBED), jnp.float32),
        jax.ShapeDtypeStruct((BATCH, EMBED), jnp.float32),
    ],
)


@functools.partial(
    pl.kernel,
    out_type=jax.ShapeDtypeStruct((BATCH,), jnp.float32),
    mesh=_mesh,
    scratch_types=[
        pltpu.VMEM((BPW,), jnp.int32),
        pltpu.VMEM((BPW,), jnp.int32),
        pltpu.VMEM((BPW, EMBED), jnp.float32),
        pltpu.VMEM((BPW, EMBED), jnp.float32),
        pltpu.VMEM((BPW, EMBED), jnp.float32),
        pltpu.VMEM((BPW, EMBED), jnp.float32),
        pltpu.VMEM((BPW,), jnp.float32),
        pltpu.SemaphoreType.DMA,
        pltpu.SemaphoreType.DMA,
    ],
    compiler_params=pltpu.CompilerParams(use_tc_tiling_on_sc=False,
                                         needs_layout_passes=False),
)
def _sc_gather_combine(uids_hbm, iids_hbm, utab_hbm, itab_hbm, pg_hbm, pa_hbm,
                       out_hbm, uidx, iidx, vtu, vti, vpg, vpa, vout, gsem,
                       psem):
    wid = lax.axis_index("s") * NUM_CORES + lax.axis_index("c")
    base = wid * BPW
    cpa = pltpu.async_copy(pa_hbm.at[pl.ds(base, BPW)], vpa, psem)
    cpg = pltpu.async_copy(pg_hbm.at[pl.ds(base, BPW)], vpg, psem)
    pltpu.sync_copy(uids_hbm.at[pl.ds(base, BPW)], uidx)
    pltpu.sync_copy(iids_hbm.at[pl.ds(base, BPW)], iidx)
    copies = []
    for k in range(NCHUNK):
        sl = pl.ds(k * GCHUNK, GCHUNK)
        copies.append(
            pltpu.async_copy(utab_hbm.at[uidx.at[sl]], vtu.at[sl], gsem))
        copies.append(
            pltpu.async_copy(itab_hbm.at[iidx.at[sl]], vti.at[sl], gsem))
    for c in copies:
        c.wait()
    cpa.wait()
    cpg.wait()

    @pl.loop(0, BPW // 16)
    def _grp(g):
        row = g * 16 + lax.iota(jnp.int32, 16)
        acc = jnp.zeros((16,), jnp.float32)
        for d in range(EMBED):
            col = jnp.full((16,), d, jnp.int32)
            uv = plsc.load_gather(vtu, [row, col])
            iv = plsc.load_gather(vti, [row, col])
            av = plsc.load_gather(vpa, [row, col])
            gv = plsc.load_gather(vpg, [row, col])
            acc += (uv + av) * (iv + gv)
        vout[pl.ds(g * 16, 16)] = 4.0 / (1.0 + jnp.exp(-acc)) + 1.0

    pltpu.sync_copy(vout, out_hbm.at[pl.ds(base, BPW)])


def kernel(user_ids, item_ids, genres_features, age_features, user_table,
           item_table, genres_W, genres_b, age_W, age_b, age_weight,
           genre_weight):
    utab, itab = _transpose_tables(user_table.T, item_table.T)
    gw = genre_weight[0] * genres_W
    gb = (genre_weight[0] * genres_b)[None, :]
    aw = age_weight[0] * age_W
    ab = (age_weight[0] * age_b)[None, :]
    pg, pa = _proj(genres_features.T, age_features.T, gw, gb, aw, ab)
    return _sc_gather_combine(user_ids, item_ids, utab, itab, pg, pa)


# speedup vs baseline: 28.6527x; 4.2470x over previous
import functools

import jax
import jax.numpy as jnp
from jax import lax
from jax.experimental import pallas as pl
from jax.experimental.pallas import tpu as pltpu
from jax.experimental.pallas import tpu_sc as plsc

BATCH = 16384
EMBED = 32
NROWS = 1000001
TPAN = 8192
NPAN = (NROWS + TPAN - 1) // TPAN
PANW = TPAN * EMBED
NUM_CORES = 2
NUM_SUBCORES = 16
NUM_WORKERS = NUM_CORES * NUM_SUBCORES
BPW = BATCH // NUM_WORKERS
GCHUNK = 128
NCHUNK = BPW // GCHUNK

_mesh = plsc.VectorSubcoreMesh(core_axis_name="c", subcore_axis_name="s")


def _repack_body(u_ref, i_ref, uo_ref, io_ref):
    u = u_ref[...]
    i = i_ref[...]
    uo_ref[...] = jnp.concatenate(
        [u[:, a * 128:(a + 1) * 128] for a in range(TPAN // 128)], axis=0)
    io_ref[...] = jnp.concatenate(
        [i[:, a * 128:(a + 1) * 128] for a in range(TPAN // 128)], axis=0)


_repack = pl.pallas_call(
    _repack_body,
    grid=(NPAN,),
    in_specs=[
        pl.BlockSpec((EMBED, TPAN), lambda i: (0, i)),
        pl.BlockSpec((EMBED, TPAN), lambda i: (0, i)),
    ],
    out_specs=[
        pl.BlockSpec((PANW // 128, 128), lambda i: (i, 0)),
        pl.BlockSpec((PANW // 128, 128), lambda i: (i, 0)),
    ],
    out_shape=[
        jax.ShapeDtypeStruct((NPAN * PANW // 128, 128), jnp.float32),
        jax.ShapeDtypeStruct((NPAN * PANW // 128, 128), jnp.float32),
    ],
)


def _proj_body(g_ref, a_ref, gw_ref, gb_ref, aw_ref, ab_ref, pg_ref, pa_ref):
    pg_ref[...] = jnp.dot(gw_ref[...], g_ref[...],
                          preferred_element_type=jnp.float32) + gb_ref[...]
    pa_ref[...] = jnp.dot(aw_ref[...], a_ref[...],
                          preferred_element_type=jnp.float32) + ab_ref[...]


_proj = pl.pallas_call(
    _proj_body,
    out_shape=[
        jax.ShapeDtypeStruct((EMBED, BATCH), jnp.float32),
        jax.ShapeDtypeStruct((EMBED, BATCH), jnp.float32),
    ],
)


@functools.partial(
    pl.kernel,
    out_type=jax.ShapeDtypeStruct((BATCH,), jnp.float32),
    mesh=_mesh,
    scratch_types=[
        pltpu.VMEM((BPW,), jnp.int32),
        pltpu.VMEM((BPW,), jnp.int32),
        pltpu.VMEM((EMBED, BPW), jnp.int32),
        pltpu.VMEM((EMBED, BPW), jnp.int32),
        pltpu.VMEM((EMBED, BPW), jnp.float32),
        pltpu.VMEM((EMBED, BPW), jnp.float32),
        pltpu.VMEM((EMBED, BPW), jnp.float32),
        pltpu.VMEM((EMBED, BPW), jnp.float32),
        pltpu.VMEM((BPW,), jnp.float32),
        pltpu.SemaphoreType.DMA,
        pltpu.SemaphoreType.DMA,
    ],
    compiler_params=pltpu.CompilerParams(use_tc_tiling_on_sc=False),
)
def _sc_gather_combine(uids_hbm, iids_hbm, utab_hbm, itab_hbm, pgT_hbm,
                       paT_hbm, out_hbm, uidx, iidx, uaddr, iaddr, vtu, vti,
                       vpg, vpa, vout, gsem, psem):
    wid = lax.axis_index("s") * NUM_CORES + lax.axis_index("c")
    base = wid * BPW
    cpa = pltpu.async_copy(paT_hbm.at[:, pl.ds(base, BPW)], vpa, psem)
    cpg = pltpu.async_copy(pgT_hbm.at[:, pl.ds(base, BPW)], vpg, psem)
    pltpu.sync_copy(uids_hbm.at[pl.ds(base, BPW)], uidx)
    pltpu.sync_copy(iids_hbm.at[pl.ds(base, BPW)], iidx)

    @pl.loop(0, EMBED)
    def _fill(d):
        wb = d * 128

        @pl.loop(0, BPW // 16)
        def _fill16(c):
            sl = pl.ds(c * 16, 16)
            u16 = uidx[sl]
            uaddr[d, sl] = ((u16 >> 13) * PANW + ((u16 >> 7) & 63) * 4096 +
                            (u16 & 127) + wb)
            i16 = iidx[sl]
            iaddr[d, sl] = ((i16 >> 13) * PANW + ((i16 >> 7) & 63) * 4096 +
                            (i16 & 127) + wb)

    window = []
    for d in range(EMBED):
        batch = []
        for k in range(NCHUNK):
            sl = pl.ds(k * GCHUNK, GCHUNK)
            batch.append(
                pltpu.async_copy(utab_hbm.at[uaddr.at[d, sl]],
                                 vtu.at[d, sl], gsem))
            batch.append(
                pltpu.async_copy(itab_hbm.at[iaddr.at[d, sl]],
                                 vti.at[d, sl], gsem))
        window.append(batch)
        if len(window) > 2:
            for c in window.pop(0):
                c.wait()
    for batch in window:
        for c in batch:
            c.wait()
    cpa.wait()
    cpg.wait()

    @pl.loop(0, BPW // 16)
    def _grp(g):
        sl = pl.ds(g * 16, 16)
        acc = jnp.zeros((16,), jnp.float32)
        for d in range(EMBED):
            acc += (vtu[d, sl] + vpa[d, sl]) * (vti[d, sl] + vpg[d, sl])
        vout[sl] = 4.0 / (1.0 + jnp.exp(-acc)) + 1.0

    pltpu.sync_copy(vout, out_hbm.at[pl.ds(base, BPW)])


def kernel(user_ids, item_ids, genres_features, age_features, user_table,
           item_table, genres_W, genres_b, age_W, age_b, age_weight,
           genre_weight):
    utab, itab = _repack(user_table.T, item_table.T)
    utab = utab.reshape(-1)
    itab = itab.reshape(-1)
    gw = (genre_weight[0] * genres_W).T
    gb = (genre_weight[0] * genres_b)[:, None]
    aw = (age_weight[0] * age_W).T
    ab = (age_weight[0] * age_b)[:, None]
    pgT, paT = _proj(genres_features.T, age_features.T, gw, gb, aw, ab)
    return _sc_gather_combine(user_ids, item_ids, utab, itab, pgT, paT)


# speedup vs baseline: 29.1334x; 1.0168x over previous
import functools

import jax
import jax.numpy as jnp
from jax import lax
from jax.experimental import pallas as pl
from jax.experimental.pallas import tpu as pltpu
from jax.experimental.pallas import tpu_sc as plsc

BATCH = 16384
EMBED = 32
NROWS = 1000001
TPAN = 8192
NPAN = (NROWS + TPAN - 1) // TPAN
PANW = TPAN * EMBED
NUM_CORES = 2
NUM_SUBCORES = 16
NUM_WORKERS = NUM_CORES * NUM_SUBCORES
BPW = BATCH // NUM_WORKERS
GCHUNK = 128
NCHUNK = BPW // GCHUNK

_mesh = plsc.VectorSubcoreMesh(core_axis_name="c", subcore_axis_name="s")


def _repack_body(u_ref, i_ref, uo_ref, io_ref):
    u = u_ref[...]
    i = i_ref[...]
    uo_ref[...] = jnp.concatenate(
        [u[:, a * 128:(a + 1) * 128] for a in range(TPAN // 128)], axis=0)
    io_ref[...] = jnp.concatenate(
        [i[:, a * 128:(a + 1) * 128] for a in range(TPAN // 128)], axis=0)


_repack = pl.pallas_call(
    _repack_body,
    grid=(NPAN,),
    in_specs=[
        pl.BlockSpec((EMBED, TPAN), lambda i: (0, i)),
        pl.BlockSpec((EMBED, TPAN), lambda i: (0, i)),
    ],
    out_specs=[
        pl.BlockSpec((PANW // 128, 128), lambda i: (i, 0)),
        pl.BlockSpec((PANW // 128, 128), lambda i: (i, 0)),
    ],
    out_shape=[
        jax.ShapeDtypeStruct((NPAN * PANW // 128, 128), jnp.float32),
        jax.ShapeDtypeStruct((NPAN * PANW // 128, 128), jnp.float32),
    ],
)


def _proj_body(g_ref, a_ref, gw_ref, gb_ref, aw_ref, ab_ref, pg_ref, pa_ref):
    pg_ref[...] = jnp.dot(gw_ref[...], g_ref[...],
                          preferred_element_type=jnp.float32) + gb_ref[...]
    pa_ref[...] = jnp.dot(aw_ref[...], a_ref[...],
                          preferred_element_type=jnp.float32) + ab_ref[...]


_proj = pl.pallas_call(
    _proj_body,
    out_shape=[
        jax.ShapeDtypeStruct((EMBED, BATCH), jnp.float32),
        jax.ShapeDtypeStruct((EMBED, BATCH), jnp.float32),
    ],
)


@functools.partial(
    pl.kernel,
    out_type=jax.ShapeDtypeStruct((BATCH,), jnp.float32),
    mesh=_mesh,
    scratch_types=[
        pltpu.VMEM((BPW,), jnp.int32),
        pltpu.VMEM((BPW,), jnp.int32),
        pltpu.VMEM((EMBED, BPW), jnp.int32),
        pltpu.VMEM((EMBED, BPW), jnp.int32),
        pltpu.VMEM((EMBED, BPW), jnp.float32),
        pltpu.VMEM((EMBED, BPW), jnp.float32),
        pltpu.VMEM((EMBED, BPW), jnp.float32),
        pltpu.VMEM((EMBED, BPW), jnp.float32),
        pltpu.VMEM((BPW,), jnp.float32),
        pltpu.SemaphoreType.DMA,
        pltpu.SemaphoreType.DMA,
    ],
    compiler_params=pltpu.CompilerParams(use_tc_tiling_on_sc=False),
)
def _sc_gather_combine(uids_hbm, iids_hbm, utab_hbm, itab_hbm, pgT_hbm,
                       paT_hbm, out_hbm, uidx, iidx, uaddr, iaddr, vtu, vti,
                       vpg, vpa, vout, gsem, psem):
    wid = lax.axis_index("s") * NUM_CORES + lax.axis_index("c")
    base = wid * BPW
    cpa = pltpu.async_copy(paT_hbm.at[:, pl.ds(base, BPW)], vpa, psem)
    cpg = pltpu.async_copy(pgT_hbm.at[:, pl.ds(base, BPW)], vpg, psem)
    pltpu.sync_copy(uids_hbm.at[pl.ds(base, BPW)], uidx)
    pltpu.sync_copy(iids_hbm.at[pl.ds(base, BPW)], iidx)

    @pl.loop(0, EMBED)
    def _fill(d):
        wb = d * 128

        @pl.loop(0, BPW // 16)
        def _fill16(c):
            sl = pl.ds(c * 16, 16)
            u16 = uidx[sl]
            uaddr[d, sl] = ((u16 >> 13) * PANW + ((u16 >> 7) & 63) * 4096 +
                            (u16 & 127) + wb)
            i16 = iidx[sl]
            iaddr[d, sl] = ((i16 >> 13) * PANW + ((i16 >> 7) & 63) * 4096 +
                            (i16 & 127) + wb)

    window = []
    for d in range(EMBED):
        batch = []
        for k in range(NCHUNK):
            sl = pl.ds(k * GCHUNK, GCHUNK)
            batch.append(
                pltpu.async_copy(utab_hbm.at[uaddr.at[d, sl]],
                                 vtu.at[d, sl], gsem))
            batch.append(
                pltpu.async_copy(itab_hbm.at[iaddr.at[d, sl]],
                                 vti.at[d, sl], gsem))
        window.append(batch)
        if len(window) > 8:
            for c in window.pop(0):
                c.wait()
    for batch in window:
        for c in batch:
            c.wait()
    cpa.wait()
    cpg.wait()

    @pl.loop(0, BPW // 16)
    def _grp(g):
        sl = pl.ds(g * 16, 16)
        acc = jnp.zeros((16,), jnp.float32)
        for d in range(EMBED):
            acc += (vtu[d, sl] + vpa[d, sl]) * (vti[d, sl] + vpg[d, sl])
        vout[sl] = 4.0 / (1.0 + jnp.exp(-acc)) + 1.0

    pltpu.sync_copy(vout, out_hbm.at[pl.ds(base, BPW)])


def kernel(user_ids, item_ids, genres_features, age_features, user_table,
           item_table, genres_W, genres_b, age_W, age_b, age_weight,
           genre_weight):
    utab, itab = _repack(user_table.T, item_table.T)
    utab = utab.reshape(-1)
    itab = itab.reshape(-1)
    gw = (genre_weight[0] * genres_W).T
    gb = (genre_weight[0] * genres_b)[:, None]
    aw = (age_weight[0] * age_W).T
    ab = (age_weight[0] * age_b)[:, None]
    pgT, paT = _proj(genres_features.T, age_features.T, gw, gb, aw, ab)
    return _sc_gather_combine(user_ids, item_ids, utab, itab, pgT, paT)
